# Initial kernel scaffold; baseline (speedup 1.0000x reference)
import jax
import jax.numpy as jnp
from jax.experimental import pallas as pl


def kernel(X, E, emb_nodes, emb_edges, edge_index, We1, be1, We2, be2, Wh1, bh1, Wh2, bh2):
    raise NotImplementedError("write your pallas kernel here")


# speedup vs baseline: 3.2529x; 3.2529x over previous
import functools

import jax
import jax.numpy as jnp
from jax import lax
from jax.experimental import pallas as pl
from jax.experimental.pallas import tpu as pltpu
from jax.experimental.pallas import tpu_sc as plsc

N_NODES = 10000
N_EDGES = 160000
D = 256
NC, NS = 2, 16
NW = NC * NS
EPW = N_EDGES // NW
EPT = N_EDGES // NS
GCH = 128
SCH = 80


def _silu(x):
    return x * jax.nn.sigmoid(x)


def _pq_body(xa_ref, wp_ref, wq_ref, p_ref, q_ref):
    xa = xa_ref[...]
    p_ref[...] = jnp.dot(xa, wp_ref[...], preferred_element_type=jnp.float32)
    q_ref[...] = jnp.dot(xa, wq_ref[...], preferred_element_type=jnp.float32)


def _node_pq(xaug, wp, wq):
    blk = 2000
    return pl.pallas_call(
        _pq_body,
        grid=(N_NODES // blk,),
        in_specs=[
            pl.BlockSpec((blk, 384), lambda i: (i, 0)),
            pl.BlockSpec((384, D), lambda i: (0, 0)),
            pl.BlockSpec((384, D), lambda i: (0, 0)),
        ],
        out_specs=[
            pl.BlockSpec((blk, D), lambda i: (i, 0)),
            pl.BlockSpec((blk, D), lambda i: (i, 0)),
        ],
        out_shape=[jax.ShapeDtypeStruct((N_NODES, D), jnp.float32)] * 2,
    )(xaug, wp, wq)


def _gather_add_body(p_hbm, q_hbm, src_hbm, dst_hbm, out_hbm,
                     idxs_v, idxd_v, pbuf, qbuf, sem1, sem2):
    c = lax.axis_index("c")
    s = lax.axis_index("s")
    w = s * NC + c
    base = pl.multiple_of(w * EPW, 8)
    pltpu.sync_copy(src_hbm.at[pl.ds(base, EPW)], idxs_v)
    pltpu.sync_copy(dst_hbm.at[pl.ds(base, EPW)], idxd_v)

    nfull = EPW // GCH
    tail = EPW - nfull * GCH

    def do_chunk(off, n):
        cp1 = pltpu.async_copy(p_hbm.at[idxs_v.at[pl.ds(off, n)]],
                               pbuf.at[pl.ds(0, n)], sem1)
        cp2 = pltpu.async_copy(q_hbm.at[idxd_v.at[pl.ds(off, n)]],
                               qbuf.at[pl.ds(0, n)], sem2)
        cp1.wait()
        cp2.wait()

        def add_row(r, _):
            for k in range(D // 16):
                sl = pl.ds(16 * k, 16)
                pbuf[r, sl] = pbuf[r, sl] + qbuf[r, sl]
            return 0

        lax.fori_loop(0, n, add_row, 0)
        pltpu.sync_copy(pbuf.at[pl.ds(0, n)],
                        out_hbm.at[pl.ds(pl.multiple_of(base + off, 8), n)])

    def chunk_body(i, _):
        do_chunk(pl.multiple_of(i * GCH, 8), GCH)
        return 0

    lax.fori_loop(0, nfull, chunk_body, 0)
    do_chunk(nfull * GCH, tail)


def _gather_add(p, q, src, dst):
    mesh = plsc.VectorSubcoreMesh(core_axis_name="c", subcore_axis_name="s",
                                  num_cores=NC, num_subcores=NS)
    return pl.kernel(
        _gather_add_body,
        out_type=jax.ShapeDtypeStruct((N_EDGES, D), jnp.float32),
        mesh=mesh,
        scratch_types=[
            pltpu.VMEM((EPW,), jnp.int32),
            pltpu.VMEM((EPW,), jnp.int32),
            pltpu.VMEM((GCH, D), jnp.float32),
            pltpu.VMEM((GCH, D), jnp.float32),
            pltpu.SemaphoreType.DMA,
            pltpu.SemaphoreType.DMA,
        ],
    )(p, q, src, dst)


def _edge_body(h_ref, e8_ref, w1d_ref, be1_ref, we2_ref, be2_ref, out_ref):
    pre = (h_ref[...]
           + jnp.dot(e8_ref[...], w1d_ref[...], preferred_element_type=jnp.float32)
           + be1_ref[...])
    h = _silu(pre)
    m = _silu(jnp.dot(h, we2_ref[...], preferred_element_type=jnp.float32)
              + be2_ref[...])
    out_ref[0] = m[:, :128]
    out_ref[1] = m[:, 128:]


def _edge_mlp(hpre, e8, w1d, be1, we2, be2):
    blk = 1000
    return pl.pallas_call(
        _edge_body,
        grid=(N_EDGES // blk,),
        in_specs=[
            pl.BlockSpec((blk, D), lambda i: (i, 0)),
            pl.BlockSpec((blk, 8), lambda i: (i, 0)),
            pl.BlockSpec((8, D), lambda i: (0, 0)),
            pl.BlockSpec((1, D), lambda i: (0, 0)),
            pl.BlockSpec((D, D), lambda i: (0, 0)),
            pl.BlockSpec((1, D), lambda i: (0, 0)),
        ],
        out_specs=pl.BlockSpec((2, blk, 128), lambda i: (0, i, 0)),
        out_shape=jax.ShapeDtypeStruct((2, N_EDGES, 128), jnp.float32),
    )(hpre, e8, w1d, be1, we2, be2)


def _scatter_body(mij2_hbm, dst2_hbm, zero_hbm, out_hbm, idx_v, buf, acc):
    c = lax.axis_index("c")
    s = lax.axis_index("s")
    rpt = 624
    niter = EPT // SCH

    roff = pl.multiple_of(s * rpt, 8)
    pltpu.sync_copy(zero_hbm.at[pl.ds(roff, rpt)], acc.at[pl.ds(roff, rpt)])

    @pl.when(s == NS - 1)
    def _():
        pltpu.sync_copy(zero_hbm.at[pl.ds(NS * rpt, N_NODES - NS * rpt)],
                        acc.at[pl.ds(NS * rpt, N_NODES - NS * rpt)])

    pltpu.sync_copy(dst2_hbm.at[s], idx_v)
    plsc.subcore_barrier()

    def body(j, _):
        off = pl.multiple_of(s * EPT + j * SCH, 8)
        pltpu.sync_copy(mij2_hbm.at[c, pl.ds(off, SCH)], buf)
        pltpu.sync_copy(buf, acc.at[idx_v.at[j]], add=True)
        return 0

    lax.fori_loop(0, niter, body, 0)
    plsc.subcore_barrier()
    pltpu.sync_copy(acc.at[pl.ds(roff, rpt)], out_hbm.at[c, pl.ds(roff, rpt)])

    @pl.when(s == NS - 1)
    def _():
        pltpu.sync_copy(acc.at[pl.ds(NS * rpt, N_NODES - NS * rpt)],
                        out_hbm.at[c, pl.ds(NS * rpt, N_NODES - NS * rpt)])


def _scatter_add(mij2, dst2, zero):
    mesh = plsc.VectorSubcoreMesh(core_axis_name="c", subcore_axis_name="s",
                                  num_cores=NC, num_subcores=NS)
    return pl.kernel(
        _scatter_body,
        out_type=jax.ShapeDtypeStruct((2, N_NODES, 128), jnp.float32),
        mesh=mesh,
        scratch_types=[
            pltpu.VMEM((EPT // SCH, SCH), jnp.int32),
            pltpu.VMEM((SCH, 128), jnp.float32),
            pltpu.VMEM_SHARED((N_NODES, 128), jnp.float32),
        ],
    )(mij2, dst2, zero)


def _node_body(x_ref, mi_ref, wa_ref, w0_ref, w1_ref, bh1_ref, wh2_ref,
               bh2_ref, out_ref):
    h2 = _silu(jnp.dot(x_ref[...], wa_ref[...], preferred_element_type=jnp.float32)
               + jnp.dot(mi_ref[0], w0_ref[...], preferred_element_type=jnp.float32)
               + jnp.dot(mi_ref[1], w1_ref[...], preferred_element_type=jnp.float32)
               + bh1_ref[...])
    out_ref[...] = (jnp.dot(h2, wh2_ref[...], preferred_element_type=jnp.float32)
                    + bh2_ref[...])


def _node_mlp(x, mi2, wh1a, wh1b0, wh1b1, bh1, wh2, bh2):
    blk = 2000
    return pl.pallas_call(
        _node_body,
        grid=(N_NODES // blk,),
        in_specs=[
            pl.BlockSpec((blk, D), lambda i: (i, 0)),
            pl.BlockSpec((2, blk, 128), lambda i: (0, i, 0)),
            pl.BlockSpec((D, D), lambda i: (0, 0)),
            pl.BlockSpec((128, D), lambda i: (0, 0)),
            pl.BlockSpec((128, D), lambda i: (0, 0)),
            pl.BlockSpec((1, D), lambda i: (0, 0)),
            pl.BlockSpec((D, D), lambda i: (0, 0)),
            pl.BlockSpec((1, D), lambda i: (0, 0)),
        ],
        out_specs=pl.BlockSpec((blk, D), lambda i: (i, 0)),
        out_shape=jax.ShapeDtypeStruct((N_NODES, D), jnp.float32),
    )(x, mi2, wh1a, wh1b0, wh1b1, bh1, wh2, bh2)


def kernel(X, E, emb_nodes, emb_edges, edge_index, We1, be1, We2, be2,
           Wh1, bh1, Wh2, bh2):
    f32 = jnp.float32
    X = X.astype(f32)
    src = edge_index[0].astype(jnp.int32)
    dst = edge_index[1].astype(jnp.int32)

    we1a = We1[:D]
    we1b = We1[D:2 * D]
    we1c = We1[2 * D:2 * D + 3]
    we1d = We1[2 * D + 3:]

    emb_pad = jnp.pad(emb_nodes.astype(f32), ((0, 0), (0, 125)))
    xaug = jnp.concatenate([X, emb_pad], axis=1)
    wp = jnp.concatenate([we1a, jnp.pad(-we1c, ((0, 125), (0, 0)))], axis=0)
    wq = jnp.concatenate([we1b, jnp.pad(we1c, ((0, 125), (0, 0)))], axis=0)

    e8 = jnp.pad(emb_edges.astype(f32), ((0, 0), (0, 3)))
    w1d8 = jnp.pad(we1d, ((0, 3), (0, 0)))

    p, q = _node_pq(xaug, wp, wq)
    hpre = _gather_add(p, q, src, dst)
    mij2 = _edge_mlp(hpre, e8, w1d8, be1.reshape(1, D), We2,
                     be2.reshape(1, D))
    dst2 = dst.reshape(NS, EPT // SCH, SCH)
    zero = jnp.zeros((N_NODES, 128), f32)
    mi2 = _scatter_add(mij2, dst2, zero)
    x_out = _node_mlp(X, mi2, Wh1[:D], Wh1[D:D + 128], Wh1[D + 128:],
                      bh1.reshape(1, D), Wh2, bh2.reshape(1, D))
    return (x_out, E, emb_nodes, emb_edges)


# speedup vs baseline: 3.7907x; 1.1653x over previous
import functools

import jax
import jax.numpy as jnp
from jax import lax
from jax.experimental import pallas as pl
from jax.experimental.pallas import tpu as pltpu
from jax.experimental.pallas import tpu_sc as plsc

N_NODES = 10000
N_EDGES = 160000
D = 256
NC, NS = 2, 16
NW = NC * NS
EPW = N_EDGES // NW
EPT = N_EDGES // NS
GCH = 64
SCH = 80


def _silu(x):
    return x * jax.nn.sigmoid(x)


def _pq_body(xa_ref, wp_ref, wq_ref, p_ref, q_ref):
    xa = xa_ref[...]
    p_ref[...] = jnp.dot(xa, wp_ref[...], preferred_element_type=jnp.float32)
    q_ref[...] = jnp.dot(xa, wq_ref[...], preferred_element_type=jnp.float32)


def _node_pq(xaug, wp, wq):
    blk = 2000
    return pl.pallas_call(
        _pq_body,
        grid=(N_NODES // blk,),
        in_specs=[
            pl.BlockSpec((blk, 384), lambda i: (i, 0)),
            pl.BlockSpec((384, D), lambda i: (0, 0)),
            pl.BlockSpec((384, D), lambda i: (0, 0)),
        ],
        out_specs=[
            pl.BlockSpec((blk, D), lambda i: (i, 0)),
            pl.BlockSpec((blk, D), lambda i: (i, 0)),
        ],
        out_shape=[jax.ShapeDtypeStruct((N_NODES, D), jnp.float32)] * 2,
    )(xaug, wp, wq)


NSLOT = 3


def _gather_add_body(p_hbm, q_hbm, src_hbm, dst_hbm, out_hbm,
                     idxs_v, idxd_v, pbuf, qbuf, semp, semq, semw):
    c = lax.axis_index("c")
    s = lax.axis_index("s")
    w = s * NC + c
    base = pl.multiple_of(w * EPW, 8)
    pltpu.sync_copy(src_hbm.at[pl.ds(base, EPW)], idxs_v)
    pltpu.sync_copy(dst_hbm.at[pl.ds(base, EPW)], idxd_v)

    nfull = EPW // GCH
    tail = EPW - nfull * GCH

    def issue(i, b):
        off = pl.multiple_of(i * GCH, 8)
        pltpu.async_copy(p_hbm.at[idxs_v.at[pl.ds(off, GCH)]],
                         pbuf.at[b], semp[b])
        pltpu.async_copy(q_hbm.at[idxd_v.at[pl.ds(off, GCH)]],
                         qbuf.at[b], semq[b])

    def wait_gather(b):
        pltpu.make_async_copy(p_hbm.at[idxs_v.at[pl.ds(0, GCH)]],
                              pbuf.at[b], semp[b]).wait()
        pltpu.make_async_copy(q_hbm.at[idxd_v.at[pl.ds(0, GCH)]],
                              qbuf.at[b], semq[b]).wait()

    def wait_wb(b):
        pltpu.make_async_copy(p_hbm.at[idxs_v.at[pl.ds(0, GCH)]],
                              pbuf.at[b], semw[b]).wait()

    def add_rows(b, n):
        pb = pbuf.at[b]
        qb = qbuf.at[b]

        def add_row(r, _):
            for k in range(D // 16):
                sl = pl.ds(16 * k, 16)
                pb[r, sl] = pb[r, sl] + qb[r, sl]
            return 0

        lax.fori_loop(0, n, add_row, 0)

    issue(0, 0)
    issue(1, 1)

    def outer(k, _):
        for b in range(NSLOT):
            i = NSLOT * k + b
            wait_gather(b)
            add_rows(b, GCH)
            pltpu.async_copy(
                pbuf.at[b],
                out_hbm.at[pl.ds(pl.multiple_of(base + i * GCH, 8), GCH)],
                semw[b])
            b2 = (b + 2) % NSLOT

            @pl.when(i >= 1)
            def _():
                wait_wb(b2)

            @pl.when(i + 2 < nfull)
            def _():
                issue(i + 2, b2)
        return 0

    lax.fori_loop(0, nfull // NSLOT, outer, 0)

    toff = nfull * GCH
    cp1 = pltpu.async_copy(p_hbm.at[idxs_v.at[pl.ds(toff, tail)]],
                           pbuf.at[0].at[pl.ds(0, tail)], semp[0])
    cp2 = pltpu.async_copy(q_hbm.at[idxd_v.at[pl.ds(toff, tail)]],
                           qbuf.at[0].at[pl.ds(0, tail)], semq[0])
    cp1.wait()
    cp2.wait()
    add_rows(0, tail)
    pltpu.sync_copy(pbuf.at[0].at[pl.ds(0, tail)],
                    out_hbm.at[pl.ds(base + toff, tail)])
    wait_wb((nfull - 1) % NSLOT)


def _gather_add(p, q, src, dst):
    mesh = plsc.VectorSubcoreMesh(core_axis_name="c", subcore_axis_name="s",
                                  num_cores=NC, num_subcores=NS)
    return pl.kernel(
        _gather_add_body,
        out_type=jax.ShapeDtypeStruct((N_EDGES, D), jnp.float32),
        mesh=mesh,
        scratch_types=[
            pltpu.VMEM((EPW,), jnp.int32),
            pltpu.VMEM((EPW,), jnp.int32),
            pltpu.VMEM((NSLOT, GCH, D), jnp.float32),
            pltpu.VMEM((NSLOT, GCH, D), jnp.float32),
            [pltpu.SemaphoreType.DMA] * NSLOT,
            [pltpu.SemaphoreType.DMA] * NSLOT,
            [pltpu.SemaphoreType.DMA] * NSLOT,
        ],
    )(p, q, src, dst)


def _edge_body(h_ref, e8_ref, w1d_ref, be1_ref, we2_ref, be2_ref, out_ref):
    pre = (h_ref[...]
           + jnp.dot(e8_ref[...], w1d_ref[...], preferred_element_type=jnp.float32)
           + be1_ref[...])
    h = _silu(pre)
    m = _silu(jnp.dot(h, we2_ref[...], preferred_element_type=jnp.float32)
              + be2_ref[...])
    out_ref[0] = m[:, :128]
    out_ref[1] = m[:, 128:]


def _edge_mlp(hpre, e8, w1d, be1, we2, be2):
    blk = 1000
    return pl.pallas_call(
        _edge_body,
        grid=(N_EDGES // blk,),
        in_specs=[
            pl.BlockSpec((blk, D), lambda i: (i, 0)),
            pl.BlockSpec((blk, 8), lambda i: (i, 0)),
            pl.BlockSpec((8, D), lambda i: (0, 0)),
            pl.BlockSpec((1, D), lambda i: (0, 0)),
            pl.BlockSpec((D, D), lambda i: (0, 0)),
            pl.BlockSpec((1, D), lambda i: (0, 0)),
        ],
        out_specs=pl.BlockSpec((2, blk, 128), lambda i: (0, i, 0)),
        out_shape=jax.ShapeDtypeStruct((2, N_EDGES, 128), jnp.float32),
    )(hpre, e8, w1d, be1, we2, be2)


SSLOT = 4


def _scatter_body(mij2_hbm, dst_hbm, zero_hbm, out_hbm, idx_v, buf, acc,
                  semi, semd, sems):
    c = lax.axis_index("c")
    s = lax.axis_index("s")
    rpt = 624
    niter = EPT // SCH
    ebase = pl.multiple_of(s * EPT, 8)

    roff = pl.multiple_of(s * rpt, 8)
    pltpu.sync_copy(zero_hbm.at[pl.ds(roff, rpt)], acc.at[pl.ds(roff, rpt)])

    @pl.when(s == NS - 1)
    def _():
        pltpu.sync_copy(zero_hbm.at[pl.ds(NS * rpt, N_NODES - NS * rpt)],
                        acc.at[pl.ds(NS * rpt, N_NODES - NS * rpt)])

    def issue_load(j, b):
        off = pl.multiple_of(ebase + j * SCH, 8)
        pltpu.async_copy(dst_hbm.at[pl.ds(off, SCH)], idx_v.at[b], semi[b])
        pltpu.async_copy(mij2_hbm.at[c, pl.ds(off, SCH)], buf.at[b], semd[b])

    def wait_load(b):
        pltpu.make_async_copy(dst_hbm.at[pl.ds(0, SCH)],
                              idx_v.at[b], semi[b]).wait()
        pltpu.make_async_copy(mij2_hbm.at[0, pl.ds(0, SCH)],
                              buf.at[b], semd[b]).wait()

    def wait_scatter(b):
        pltpu.make_async_copy(mij2_hbm.at[0, pl.ds(0, SCH)],
                              buf.at[b], sems[b]).wait()

    issue_load(0, 0)
    plsc.subcore_barrier()

    def step(j, b):
        wait_load(b)
        pltpu.async_copy(buf.at[b], acc.at[idx_v.at[b]], sems[b], add=True)
        nb = (b + 1) % SSLOT

        @pl.when(j >= SSLOT - 1)
        def _():
            wait_scatter(nb)

        @pl.when(j + 1 < niter)
        def _():
            issue_load(j + 1, nb)

    def outer(k, _):
        for b in range(SSLOT):
            step(SSLOT * k + b, b)
        return 0

    lax.fori_loop(0, niter // SSLOT, outer, 0)
    step(niter - 1, (niter - 1) % SSLOT)
    for b in range(SSLOT):
        if b != ((niter - 1) + 1) % SSLOT:
            wait_scatter(b)

    plsc.subcore_barrier()
    pltpu.sync_copy(acc.at[pl.ds(roff, rpt)], out_hbm.at[c, pl.ds(roff, rpt)])

    @pl.when(s == NS - 1)
    def _():
        pltpu.sync_copy(acc.at[pl.ds(NS * rpt, N_NODES - NS * rpt)],
                        out_hbm.at[c, pl.ds(NS * rpt, N_NODES - NS * rpt)])


def _scatter_add(mij2, dst, zero):
    mesh = plsc.VectorSubcoreMesh(core_axis_name="c", subcore_axis_name="s",
                                  num_cores=NC, num_subcores=NS)
    return pl.kernel(
        _scatter_body,
        out_type=jax.ShapeDtypeStruct((2, N_NODES, 128), jnp.float32),
        mesh=mesh,
        scratch_types=[
            pltpu.VMEM((SSLOT, SCH), jnp.int32),
            pltpu.VMEM((SSLOT, SCH, 128), jnp.float32),
            pltpu.VMEM_SHARED((N_NODES, 128), jnp.float32),
            [pltpu.SemaphoreType.DMA] * SSLOT,
            [pltpu.SemaphoreType.DMA] * SSLOT,
            [pltpu.SemaphoreType.DMA] * SSLOT,
        ],
    )(mij2, dst, zero)


def _node_body(x_ref, mi_ref, wa_ref, w0_ref, w1_ref, bh1_ref, wh2_ref,
               bh2_ref, out_ref):
    h2 = _silu(jnp.dot(x_ref[...], wa_ref[...], preferred_element_type=jnp.float32)
               + jnp.dot(mi_ref[0], w0_ref[...], preferred_element_type=jnp.float32)
               + jnp.dot(mi_ref[1], w1_ref[...], preferred_element_type=jnp.float32)
               + bh1_ref[...])
    out_ref[...] = (jnp.dot(h2, wh2_ref[...], preferred_element_type=jnp.float32)
                    + bh2_ref[...])


def _node_mlp(x, mi2, wh1a, wh1b0, wh1b1, bh1, wh2, bh2):
    blk = 2000
    return pl.pallas_call(
        _node_body,
        grid=(N_NODES // blk,),
        in_specs=[
            pl.BlockSpec((blk, D), lambda i: (i, 0)),
            pl.BlockSpec((2, blk, 128), lambda i: (0, i, 0)),
            pl.BlockSpec((D, D), lambda i: (0, 0)),
            pl.BlockSpec((128, D), lambda i: (0, 0)),
            pl.BlockSpec((128, D), lambda i: (0, 0)),
            pl.BlockSpec((1, D), lambda i: (0, 0)),
            pl.BlockSpec((D, D), lambda i: (0, 0)),
            pl.BlockSpec((1, D), lambda i: (0, 0)),
        ],
        out_specs=pl.BlockSpec((blk, D), lambda i: (i, 0)),
        out_shape=jax.ShapeDtypeStruct((N_NODES, D), jnp.float32),
    )(x, mi2, wh1a, wh1b0, wh1b1, bh1, wh2, bh2)


def kernel(X, E, emb_nodes, emb_edges, edge_index, We1, be1, We2, be2,
           Wh1, bh1, Wh2, bh2):
    f32 = jnp.float32
    X = X.astype(f32)
    src = edge_index[0].astype(jnp.int32)
    dst = edge_index[1].astype(jnp.int32)

    we1a = We1[:D]
    we1b = We1[D:2 * D]
    we1c = We1[2 * D:2 * D + 3]
    we1d = We1[2 * D + 3:]

    emb_pad = jnp.pad(emb_nodes.astype(f32), ((0, 0), (0, 125)))
    xaug = jnp.concatenate([X, emb_pad], axis=1)
    wp = jnp.concatenate([we1a, jnp.pad(-we1c, ((0, 125), (0, 0)))], axis=0)
    wq = jnp.concatenate([we1b, jnp.pad(we1c, ((0, 125), (0, 0)))], axis=0)

    e8 = jnp.pad(emb_edges.astype(f32), ((0, 0), (0, 3)))
    w1d8 = jnp.pad(we1d, ((0, 3), (0, 0)))

    p, q = _node_pq(xaug, wp, wq)
    hpre = _gather_add(p, q, src, dst)
    mij2 = _edge_mlp(hpre, e8, w1d8, be1.reshape(1, D), We2,
                     be2.reshape(1, D))
    zero = jnp.zeros((N_NODES, 128), f32)
    mi2 = _scatter_add(mij2, dst, zero)
    x_out = _node_mlp(X, mi2, Wh1[:D], Wh1[D:D + 128], Wh1[D + 128:],
                      bh1.reshape(1, D), Wh2, bh2.reshape(1, D))
    return (x_out, E, emb_nodes, emb_edges)


# speedup vs baseline: 4.1621x; 1.0980x over previous
import jax
import jax.numpy as jnp
from jax import lax
from jax.experimental import pallas as pl
from jax.experimental.pallas import tpu as pltpu
from jax.experimental.pallas import tpu_sc as plsc

N_NODES = 10000
N_EDGES = 160000
D = 256
DI = 128
NC, NS = 2, 16
NW = NC * NS
EPW = N_EDGES // NW
EPT = N_EDGES // NS
GCH = 128
SCH = 80
NSLOT = 3
SSLOT = 4


def _silu(x):
    return x * jax.nn.sigmoid(x)


def _pack_bf16(x):
    b = lax.bitcast_convert_type(x, jnp.uint32)
    r = b + jnp.uint32(0x7FFF) + ((b >> 16) & jnp.uint32(1))
    hi = r[:, :DI] & jnp.uint32(0xFFFF0000)
    lo = r[:, DI:] >> 16
    return lax.bitcast_convert_type(hi | lo, jnp.int32)


def _unpack_bf16(w_i32):
    w = lax.bitcast_convert_type(w_i32, jnp.uint32)
    hi = lax.bitcast_convert_type(w & jnp.uint32(0xFFFF0000), jnp.float32)
    lo = lax.bitcast_convert_type(w << 16, jnp.float32)
    return jnp.concatenate([hi, lo], axis=1)


def _pq_body(x_ref, em_ref, wa_ref, wcp_ref, wb_ref, wcq_ref, p_ref, q_ref):
    x = x_ref[...]
    em = em_ref[...]
    p = (jnp.dot(x, wa_ref[...], preferred_element_type=jnp.float32)
         + jnp.dot(em, wcp_ref[...], preferred_element_type=jnp.float32))
    q = (jnp.dot(x, wb_ref[...], preferred_element_type=jnp.float32)
         + jnp.dot(em, wcq_ref[...], preferred_element_type=jnp.float32))
    p_ref[...] = _pack_bf16(p)
    q_ref[...] = _pack_bf16(q)


def _node_pq(x, em8, wa, wcp, wb, wcq):
    blk = 2000
    return pl.pallas_call(
        _pq_body,
        grid=(N_NODES // blk,),
        in_specs=[
            pl.BlockSpec((blk, D), lambda i: (i, 0)),
            pl.BlockSpec((blk, 8), lambda i: (i, 0)),
            pl.BlockSpec((D, D), lambda i: (0, 0)),
            pl.BlockSpec((8, D), lambda i: (0, 0)),
            pl.BlockSpec((D, D), lambda i: (0, 0)),
            pl.BlockSpec((8, D), lambda i: (0, 0)),
        ],
        out_specs=[
            pl.BlockSpec((blk, DI), lambda i: (i, 0)),
            pl.BlockSpec((blk, DI), lambda i: (i, 0)),
        ],
        out_shape=[jax.ShapeDtypeStruct((N_NODES, DI), jnp.int32)] * 2,
    )(x, em8, wa, wcp, wb, wcq)


def _gather_body(p_hbm, q_hbm, src_hbm, dst_hbm, ps_hbm, qd_hbm,
                 idxs_v, idxd_v, pbuf, qbuf, semp, semq, semwp, semwq):
    c = lax.axis_index("c")
    s = lax.axis_index("s")
    w = s * NC + c
    base = pl.multiple_of(w * EPW, 8)
    pltpu.sync_copy(src_hbm.at[pl.ds(base, EPW)], idxs_v)
    pltpu.sync_copy(dst_hbm.at[pl.ds(base, EPW)], idxd_v)

    nfull = EPW // GCH
    tail = EPW - nfull * GCH

    def issue(i, b):
        off = pl.multiple_of(i * GCH, 8)
        pltpu.async_copy(p_hbm.at[idxs_v.at[pl.ds(off, GCH)]],
                         pbuf.at[b], semp[b])
        pltpu.async_copy(q_hbm.at[idxd_v.at[pl.ds(off, GCH)]],
                         qbuf.at[b], semq[b])

    def wait_gather(b):
        pltpu.make_async_copy(p_hbm.at[idxs_v.at[pl.ds(0, GCH)]],
                              pbuf.at[b], semp[b]).wait()
        pltpu.make_async_copy(q_hbm.at[idxd_v.at[pl.ds(0, GCH)]],
                              qbuf.at[b], semq[b]).wait()

    def wb(i, b):
        off = pl.multiple_of(base + i * GCH, 8)
        pltpu.async_copy(pbuf.at[b], ps_hbm.at[pl.ds(off, GCH)], semwp[b])
        pltpu.async_copy(qbuf.at[b], qd_hbm.at[pl.ds(off, GCH)], semwq[b])

    def wait_wb(b):
        pltpu.make_async_copy(p_hbm.at[idxs_v.at[pl.ds(0, GCH)]],
                              pbuf.at[b], semwp[b]).wait()
        pltpu.make_async_copy(q_hbm.at[idxd_v.at[pl.ds(0, GCH)]],
                              qbuf.at[b], semwq[b]).wait()

    issue(0, 0)
    issue(1, 1)

    def outer(k, _):
        for b in range(NSLOT):
            i = NSLOT * k + b
            wait_gather(b)
            wb(i, b)
            b2 = (b + 2) % NSLOT

            @pl.when(i >= 1)
            def _():
                wait_wb(b2)

            @pl.when(i + 2 < nfull)
            def _():
                issue(i + 2, b2)
        return 0

    lax.fori_loop(0, nfull // NSLOT, outer, 0)

    toff = nfull * GCH
    cp1 = pltpu.async_copy(p_hbm.at[idxs_v.at[pl.ds(toff, tail)]],
                           pbuf.at[0].at[pl.ds(0, tail)], semp[0])
    cp2 = pltpu.async_copy(q_hbm.at[idxd_v.at[pl.ds(toff, tail)]],
                           qbuf.at[0].at[pl.ds(0, tail)], semq[0])
    cp1.wait()
    cp2.wait()
    pltpu.sync_copy(pbuf.at[0].at[pl.ds(0, tail)],
                    ps_hbm.at[pl.ds(base + toff, tail)])
    pltpu.sync_copy(qbuf.at[0].at[pl.ds(0, tail)],
                    qd_hbm.at[pl.ds(base + toff, tail)])
    wait_wb((nfull - 1) % NSLOT)


def _gather_pq(p, q, src, dst):
    mesh = plsc.VectorSubcoreMesh(core_axis_name="c", subcore_axis_name="s",
                                  num_cores=NC, num_subcores=NS)
    return pl.kernel(
        _gather_body,
        out_type=[jax.ShapeDtypeStruct((N_EDGES, DI), jnp.int32)] * 2,
        mesh=mesh,
        scratch_types=[
            pltpu.VMEM((EPW,), jnp.int32),
            pltpu.VMEM((EPW,), jnp.int32),
            pltpu.VMEM((NSLOT, GCH, DI), jnp.int32),
            pltpu.VMEM((NSLOT, GCH, DI), jnp.int32),
            [pltpu.SemaphoreType.DMA] * NSLOT,
            [pltpu.SemaphoreType.DMA] * NSLOT,
            [pltpu.SemaphoreType.DMA] * NSLOT,
            [pltpu.SemaphoreType.DMA] * NSLOT,
        ],
    )(p, q, src, dst)


def _edge_body(ps_ref, qd_ref, e5_ref, w1d_ref, be1_ref, we2_ref, be2_ref,
               out_ref):
    pre = (_unpack_bf16(ps_ref[...]) + _unpack_bf16(qd_ref[...])
           + jnp.dot(e5_ref[...], w1d_ref[...],
                     preferred_element_type=jnp.float32)
           + be1_ref[...])
    h = _silu(pre)
    m = _silu(jnp.dot(h.astype(jnp.bfloat16), we2_ref[...],
                      preferred_element_type=jnp.float32)
              + be2_ref[...])
    out_ref[0] = m[:, :128]
    out_ref[1] = m[:, 128:]


def _edge_mlp(ps, qd, e5, w1d, be1, we2_bf, be2):
    blk = 1000
    return pl.pallas_call(
        _edge_body,
        grid=(N_EDGES // blk,),
        in_specs=[
            pl.BlockSpec((blk, DI), lambda i: (i, 0)),
            pl.BlockSpec((blk, DI), lambda i: (i, 0)),
            pl.BlockSpec((blk, 8), lambda i: (i, 0)),
            pl.BlockSpec((8, D), lambda i: (0, 0)),
            pl.BlockSpec((1, D), lambda i: (0, 0)),
            pl.BlockSpec((D, D), lambda i: (0, 0)),
            pl.BlockSpec((1, D), lambda i: (0, 0)),
        ],
        out_specs=pl.BlockSpec((2, blk, 128), lambda i: (0, i, 0)),
        out_shape=jax.ShapeDtypeStruct((2, N_EDGES, 128), jnp.float32),
    )(ps, qd, e5, w1d, be1, we2_bf, be2)


def _scatter_body(mij2_hbm, dst_hbm, zero_hbm, out_hbm, idx_v, buf, acc,
                  semi, semd, sems):
    c = lax.axis_index("c")
    s = lax.axis_index("s")
    rpt = 624
    niter = EPT // SCH
    ebase = pl.multiple_of(s * EPT, 8)

    roff = pl.multiple_of(s * rpt, 8)
    pltpu.sync_copy(zero_hbm.at[pl.ds(roff, rpt)], acc.at[pl.ds(roff, rpt)])

    @pl.when(s == NS - 1)
    def _():
        pltpu.sync_copy(zero_hbm.at[pl.ds(NS * rpt, N_NODES - NS * rpt)],
                        acc.at[pl.ds(NS * rpt, N_NODES - NS * rpt)])

    def issue_load(j, b):
        off = pl.multiple_of(ebase + j * SCH, 8)
        pltpu.async_copy(dst_hbm.at[pl.ds(off, SCH)], idx_v.at[b], semi[b])
        pltpu.async_copy(mij2_hbm.at[c, pl.ds(off, SCH)], buf.at[b], semd[b])

    def wait_load(b):
        pltpu.make_async_copy(dst_hbm.at[pl.ds(0, SCH)],
                              idx_v.at[b], semi[b]).wait()
        pltpu.make_async_copy(mij2_hbm.at[0, pl.ds(0, SCH)],
                              buf.at[b], semd[b]).wait()

    def wait_scatter(b):
        pltpu.make_async_copy(mij2_hbm.at[0, pl.ds(0, SCH)],
                              buf.at[b], sems[b]).wait()

    issue_load(0, 0)
    plsc.subcore_barrier()

    def step(j, b):
        wait_load(b)
        pltpu.async_copy(buf.at[b], acc.at[idx_v.at[b]], sems[b], add=True)
        nb = (b + 1) % SSLOT

        @pl.when(j >= SSLOT - 1)
        def _():
            wait_scatter(nb)

        @pl.when(j + 1 < niter)
        def _():
            issue_load(j + 1, nb)

    def outer(k, _):
        for b in range(SSLOT):
            step(SSLOT * k + b, b)
        return 0

    lax.fori_loop(0, niter // SSLOT, outer, 0)
    step(niter - 1, (niter - 1) % SSLOT)
    for b in range(SSLOT):
        if b != ((niter - 1) + 1) % SSLOT:
            wait_scatter(b)

    plsc.subcore_barrier()
    pltpu.sync_copy(acc.at[pl.ds(roff, rpt)], out_hbm.at[c, pl.ds(roff, rpt)])

    @pl.when(s == NS - 1)
    def _():
        pltpu.sync_copy(acc.at[pl.ds(NS * rpt, N_NODES - NS * rpt)],
                        out_hbm.at[c, pl.ds(NS * rpt, N_NODES - NS * rpt)])


def _scatter_add(mij2, dst, zero):
    mesh = plsc.VectorSubcoreMesh(core_axis_name="c", subcore_axis_name="s",
                                  num_cores=NC, num_subcores=NS)
    return pl.kernel(
        _scatter_body,
        out_type=jax.ShapeDtypeStruct((2, N_NODES, 128), jnp.float32),
        mesh=mesh,
        scratch_types=[
            pltpu.VMEM((SSLOT, SCH), jnp.int32),
            pltpu.VMEM((SSLOT, SCH, 128), jnp.float32),
            pltpu.VMEM_SHARED((N_NODES, 128), jnp.float32),
            [pltpu.SemaphoreType.DMA] * SSLOT,
            [pltpu.SemaphoreType.DMA] * SSLOT,
            [pltpu.SemaphoreType.DMA] * SSLOT,
        ],
    )(mij2, dst, zero)


def _node_body(x_ref, mi_ref, wa_ref, w0_ref, w1_ref, bh1_ref, wh2_ref,
               bh2_ref, out_ref):
    h2 = _silu(jnp.dot(x_ref[...], wa_ref[...], preferred_element_type=jnp.float32)
               + jnp.dot(mi_ref[0], w0_ref[...], preferred_element_type=jnp.float32)
               + jnp.dot(mi_ref[1], w1_ref[...], preferred_element_type=jnp.float32)
               + bh1_ref[...])
    out_ref[...] = (jnp.dot(h2, wh2_ref[...], preferred_element_type=jnp.float32)
                    + bh2_ref[...])


def _node_mlp(x, mi2, wh1a, wh1b0, wh1b1, bh1, wh2, bh2):
    blk = 2000
    return pl.pallas_call(
        _node_body,
        grid=(N_NODES // blk,),
        in_specs=[
            pl.BlockSpec((blk, D), lambda i: (i, 0)),
            pl.BlockSpec((2, blk, 128), lambda i: (0, i, 0)),
            pl.BlockSpec((D, D), lambda i: (0, 0)),
            pl.BlockSpec((128, D), lambda i: (0, 0)),
            pl.BlockSpec((128, D), lambda i: (0, 0)),
            pl.BlockSpec((1, D), lambda i: (0, 0)),
            pl.BlockSpec((D, D), lambda i: (0, 0)),
            pl.BlockSpec((1, D), lambda i: (0, 0)),
        ],
        out_specs=pl.BlockSpec((blk, D), lambda i: (i, 0)),
        out_shape=jax.ShapeDtypeStruct((N_NODES, D), jnp.float32),
    )(x, mi2, wh1a, wh1b0, wh1b1, bh1, wh2, bh2)


def kernel(X, E, emb_nodes, emb_edges, edge_index, We1, be1, We2, be2,
           Wh1, bh1, Wh2, bh2):
    f32 = jnp.float32
    X = X.astype(f32)
    src = edge_index[0].astype(jnp.int32)
    dst = edge_index[1].astype(jnp.int32)

    we1a = We1[:D]
    we1b = We1[D:2 * D]
    we1c = jnp.pad(We1[2 * D:2 * D + 3], ((0, 5), (0, 0)))
    we1d = jnp.pad(We1[2 * D + 3:], ((0, 3), (0, 0)))
    em8 = jnp.pad(emb_nodes.astype(f32), ((0, 0), (0, 5)))
    e8 = jnp.pad(emb_edges.astype(f32), ((0, 0), (0, 3)))

    p, q = _node_pq(X, em8, we1a, -we1c, we1b, we1c)
    ps, qd = _gather_pq(p, q, src, dst)
    mij2 = _edge_mlp(ps, qd, e8, we1d, be1.reshape(1, D),
                     We2.astype(jnp.bfloat16), be2.reshape(1, D))
    zero = jnp.zeros((N_NODES, 128), f32)
    mi2 = _scatter_add(mij2, dst, zero)
    x_out = _node_mlp(X, mi2, Wh1[:D], Wh1[D:D + 128], Wh1[D + 128:],
                      bh1.reshape(1, D), Wh2, bh2.reshape(1, D))
    return (x_out, E, emb_nodes, emb_edges)


# speedup vs baseline: 4.5121x; 1.0841x over previous
import jax
import jax.numpy as jnp
from jax import lax
from jax.experimental import pallas as pl
from jax.experimental.pallas import tpu as pltpu
from jax.experimental.pallas import tpu_sc as plsc

N_NODES = 10000
N_EDGES = 160000
D = 256
DI = 128
NC, NS = 2, 16
NW = NC * NS
EPW = N_EDGES // NW
EPT = N_EDGES // NS
GCH = 128
SCH = 80
NSLOT = 3
SSLOT = 4


def _silu(x):
    return x * jax.nn.sigmoid(x)


def _pack_bf16(x):
    b = lax.bitcast_convert_type(x, jnp.uint32)
    r = b + jnp.uint32(0x7FFF) + ((b >> 16) & jnp.uint32(1))
    hi = r[:, :DI] & jnp.uint32(0xFFFF0000)
    lo = r[:, DI:] >> 16
    return lax.bitcast_convert_type(hi | lo, jnp.int32)


def _unpack_bf16(w_i32):
    w = lax.bitcast_convert_type(w_i32, jnp.uint32)
    hi = lax.bitcast_convert_type(w & jnp.uint32(0xFFFF0000), jnp.float32)
    lo = lax.bitcast_convert_type(w << 16, jnp.float32)
    return jnp.concatenate([hi, lo], axis=1)


def _pq_body(x_ref, em_ref, wa_ref, wcp_ref, wb_ref, wcq_ref, p_ref, q_ref):
    x = x_ref[...]
    em = em_ref[...]
    p = (jnp.dot(x, wa_ref[...], preferred_element_type=jnp.float32)
         + jnp.dot(em, wcp_ref[...], preferred_element_type=jnp.float32))
    q = (jnp.dot(x, wb_ref[...], preferred_element_type=jnp.float32)
         + jnp.dot(em, wcq_ref[...], preferred_element_type=jnp.float32))
    p_ref[...] = _pack_bf16(p)
    q_ref[...] = _pack_bf16(q)


def _node_pq(x, em8, wa, wcp, wb, wcq):
    blk = 2000
    return pl.pallas_call(
        _pq_body,
        grid=(N_NODES // blk,),
        in_specs=[
            pl.BlockSpec((blk, D), lambda i: (i, 0)),
            pl.BlockSpec((blk, 8), lambda i: (i, 0)),
            pl.BlockSpec((D, D), lambda i: (0, 0)),
            pl.BlockSpec((8, D), lambda i: (0, 0)),
            pl.BlockSpec((D, D), lambda i: (0, 0)),
            pl.BlockSpec((8, D), lambda i: (0, 0)),
        ],
        out_specs=[
            pl.BlockSpec((blk, DI), lambda i: (i, 0)),
            pl.BlockSpec((blk, DI), lambda i: (i, 0)),
        ],
        out_shape=[jax.ShapeDtypeStruct((N_NODES, DI), jnp.int32)] * 2,
    )(x, em8, wa, wcp, wb, wcq)


def _gather_body(p_hbm, q_hbm, src_hbm, dst_hbm, ps_hbm, qd_hbm,
                 idxs_v, idxd_v, pbuf, qbuf, semp, semq, semwp, semwq):
    c = lax.axis_index("c")
    s = lax.axis_index("s")
    w = s * NC + c
    base = pl.multiple_of(w * EPW, 8)
    pltpu.sync_copy(src_hbm.at[pl.ds(base, EPW)], idxs_v)
    pltpu.sync_copy(dst_hbm.at[pl.ds(base, EPW)], idxd_v)

    nfull = EPW // GCH
    tail = EPW - nfull * GCH

    def issue(i, b):
        off = pl.multiple_of(i * GCH, 8)
        pltpu.async_copy(p_hbm.at[idxs_v.at[pl.ds(off, GCH)]],
                         pbuf.at[b], semp[b])
        pltpu.async_copy(q_hbm.at[idxd_v.at[pl.ds(off, GCH)]],
                         qbuf.at[b], semq[b])

    def wait_gather(b):
        pltpu.make_async_copy(p_hbm.at[idxs_v.at[pl.ds(0, GCH)]],
                              pbuf.at[b], semp[b]).wait()
        pltpu.make_async_copy(q_hbm.at[idxd_v.at[pl.ds(0, GCH)]],
                              qbuf.at[b], semq[b]).wait()

    def wb(i, b):
        off = pl.multiple_of(base + i * GCH, 8)
        pltpu.async_copy(pbuf.at[b], ps_hbm.at[pl.ds(off, GCH)], semwp[b])
        pltpu.async_copy(qbuf.at[b], qd_hbm.at[pl.ds(off, GCH)], semwq[b])

    def wait_wb(b):
        pltpu.make_async_copy(p_hbm.at[idxs_v.at[pl.ds(0, GCH)]],
                              pbuf.at[b], semwp[b]).wait()
        pltpu.make_async_copy(q_hbm.at[idxd_v.at[pl.ds(0, GCH)]],
                              qbuf.at[b], semwq[b]).wait()

    issue(0, 0)
    issue(1, 1)

    def outer(k, _):
        for b in range(NSLOT):
            i = NSLOT * k + b
            wait_gather(b)
            wb(i, b)
            b2 = (b + 2) % NSLOT

            @pl.when(i >= 1)
            def _():
                wait_wb(b2)

            @pl.when(i + 2 < nfull)
            def _():
                issue(i + 2, b2)
        return 0

    lax.fori_loop(0, nfull // NSLOT, outer, 0)

    toff = nfull * GCH
    cp1 = pltpu.async_copy(p_hbm.at[idxs_v.at[pl.ds(toff, tail)]],
                           pbuf.at[0].at[pl.ds(0, tail)], semp[0])
    cp2 = pltpu.async_copy(q_hbm.at[idxd_v.at[pl.ds(toff, tail)]],
                           qbuf.at[0].at[pl.ds(0, tail)], semq[0])
    cp1.wait()
    cp2.wait()
    pltpu.sync_copy(pbuf.at[0].at[pl.ds(0, tail)],
                    ps_hbm.at[pl.ds(base + toff, tail)])
    pltpu.sync_copy(qbuf.at[0].at[pl.ds(0, tail)],
                    qd_hbm.at[pl.ds(base + toff, tail)])
    wait_wb((nfull - 1) % NSLOT)


def _gather_pq(p, q, src, dst):
    mesh = plsc.VectorSubcoreMesh(core_axis_name="c", subcore_axis_name="s",
                                  num_cores=NC, num_subcores=NS)
    return pl.kernel(
        _gather_body,
        out_type=[jax.ShapeDtypeStruct((N_EDGES, DI), jnp.int32)] * 2,
        mesh=mesh,
        scratch_types=[
            pltpu.VMEM((EPW,), jnp.int32),
            pltpu.VMEM((EPW,), jnp.int32),
            pltpu.VMEM((NSLOT, GCH, DI), jnp.int32),
            pltpu.VMEM((NSLOT, GCH, DI), jnp.int32),
            [pltpu.SemaphoreType.DMA] * NSLOT,
            [pltpu.SemaphoreType.DMA] * NSLOT,
            [pltpu.SemaphoreType.DMA] * NSLOT,
            [pltpu.SemaphoreType.DMA] * NSLOT,
        ],
    )(p, q, src, dst)


def _edge_body(ps_ref, qd_ref, e5_ref, w1d_ref, be1_ref, we2_ref, be2_ref,
               out_ref):
    pre = (_unpack_bf16(ps_ref[...]) + _unpack_bf16(qd_ref[...])
           + jnp.dot(e5_ref[...], w1d_ref[...],
                     preferred_element_type=jnp.float32)
           + be1_ref[...])
    h = _silu(pre)
    m = _silu(jnp.dot(h.astype(jnp.bfloat16), we2_ref[...],
                      preferred_element_type=jnp.float32)
              + be2_ref[...])
    out_ref[0] = m[:, :128]
    out_ref[1] = m[:, 128:]


def _edge_mlp(ps, qd, e5, w1d, be1, we2_bf, be2):
    blk = 1000
    return pl.pallas_call(
        _edge_body,
        grid=(N_EDGES // blk,),
        in_specs=[
            pl.BlockSpec((blk, DI), lambda i: (i, 0)),
            pl.BlockSpec((blk, DI), lambda i: (i, 0)),
            pl.BlockSpec((blk, 5), lambda i: (i, 0)),
            pl.BlockSpec((5, D), lambda i: (0, 0)),
            pl.BlockSpec((1, D), lambda i: (0, 0)),
            pl.BlockSpec((D, D), lambda i: (0, 0)),
            pl.BlockSpec((1, D), lambda i: (0, 0)),
        ],
        out_specs=pl.BlockSpec((2, blk, 128), lambda i: (0, i, 0)),
        out_shape=jax.ShapeDtypeStruct((2, N_EDGES, 128), jnp.float32),
    )(ps, qd, e5, w1d, be1, we2_bf, be2)


def _scatter_body(mij2_hbm, dst_hbm, zero_hbm, out_hbm, idx_v, buf, acc,
                  semi, semd, sems):
    c = lax.axis_index("c")
    s = lax.axis_index("s")
    rpt = 624
    niter = EPT // SCH
    ebase = pl.multiple_of(s * EPT, 8)

    roff = pl.multiple_of(s * rpt, 8)
    pltpu.sync_copy(zero_hbm.at[pl.ds(roff, rpt)], acc.at[pl.ds(roff, rpt)])

    @pl.when(s == NS - 1)
    def _():
        pltpu.sync_copy(zero_hbm.at[pl.ds(NS * rpt, N_NODES - NS * rpt)],
                        acc.at[pl.ds(NS * rpt, N_NODES - NS * rpt)])

    def issue_load(j, b):
        off = pl.multiple_of(ebase + j * SCH, 8)
        pltpu.async_copy(dst_hbm.at[pl.ds(off, SCH)], idx_v.at[b], semi[b])
        pltpu.async_copy(mij2_hbm.at[c, pl.ds(off, SCH)], buf.at[b], semd[b])

    def wait_load(b):
        pltpu.make_async_copy(dst_hbm.at[pl.ds(0, SCH)],
                              idx_v.at[b], semi[b]).wait()
        pltpu.make_async_copy(mij2_hbm.at[0, pl.ds(0, SCH)],
                              buf.at[b], semd[b]).wait()

    def wait_scatter(b):
        pltpu.make_async_copy(mij2_hbm.at[0, pl.ds(0, SCH)],
                              buf.at[b], sems[b]).wait()

    issue_load(0, 0)
    plsc.subcore_barrier()

    def step(j, b):
        wait_load(b)
        pltpu.async_copy(buf.at[b], acc.at[idx_v.at[b]], sems[b], add=True)
        nb = (b + 1) % SSLOT

        @pl.when(j >= SSLOT - 1)
        def _():
            wait_scatter(nb)

        @pl.when(j + 1 < niter)
        def _():
            issue_load(j + 1, nb)

    def outer(k, _):
        for b in range(SSLOT):
            step(SSLOT * k + b, b)
        return 0

    lax.fori_loop(0, niter // SSLOT, outer, 0)
    step(niter - 1, (niter - 1) % SSLOT)
    for b in range(SSLOT):
        if b != ((niter - 1) + 1) % SSLOT:
            wait_scatter(b)

    plsc.subcore_barrier()
    pltpu.sync_copy(acc.at[pl.ds(roff, rpt)], out_hbm.at[c, pl.ds(roff, rpt)])

    @pl.when(s == NS - 1)
    def _():
        pltpu.sync_copy(acc.at[pl.ds(NS * rpt, N_NODES - NS * rpt)],
                        out_hbm.at[c, pl.ds(NS * rpt, N_NODES - NS * rpt)])


def _scatter_add(mij2, dst, zero):
    mesh = plsc.VectorSubcoreMesh(core_axis_name="c", subcore_axis_name="s",
                                  num_cores=NC, num_subcores=NS)
    return pl.kernel(
        _scatter_body,
        out_type=jax.ShapeDtypeStruct((2, N_NODES, 128), jnp.float32),
        mesh=mesh,
        scratch_types=[
            pltpu.VMEM((SSLOT, SCH), jnp.int32),
            pltpu.VMEM((SSLOT, SCH, 128), jnp.float32),
            pltpu.VMEM_SHARED((N_NODES, 128), jnp.float32),
            [pltpu.SemaphoreType.DMA] * SSLOT,
            [pltpu.SemaphoreType.DMA] * SSLOT,
            [pltpu.SemaphoreType.DMA] * SSLOT,
        ],
    )(mij2, dst, zero)


def _node_body(x_ref, mi_ref, wa_ref, w0_ref, w1_ref, bh1_ref, wh2_ref,
               bh2_ref, out_ref):
    h2 = _silu(jnp.dot(x_ref[...], wa_ref[...], preferred_element_type=jnp.float32)
               + jnp.dot(mi_ref[0], w0_ref[...], preferred_element_type=jnp.float32)
               + jnp.dot(mi_ref[1], w1_ref[...], preferred_element_type=jnp.float32)
               + bh1_ref[...])
    out_ref[...] = (jnp.dot(h2, wh2_ref[...], preferred_element_type=jnp.float32)
                    + bh2_ref[...])


def _node_mlp(x, mi2, wh1a, wh1b0, wh1b1, bh1, wh2, bh2):
    blk = 2000
    return pl.pallas_call(
        _node_body,
        grid=(N_NODES // blk,),
        in_specs=[
            pl.BlockSpec((blk, D), lambda i: (i, 0)),
            pl.BlockSpec((2, blk, 128), lambda i: (0, i, 0)),
            pl.BlockSpec((D, D), lambda i: (0, 0)),
            pl.BlockSpec((128, D), lambda i: (0, 0)),
            pl.BlockSpec((128, D), lambda i: (0, 0)),
            pl.BlockSpec((1, D), lambda i: (0, 0)),
            pl.BlockSpec((D, D), lambda i: (0, 0)),
            pl.BlockSpec((1, D), lambda i: (0, 0)),
        ],
        out_specs=pl.BlockSpec((blk, D), lambda i: (i, 0)),
        out_shape=jax.ShapeDtypeStruct((N_NODES, D), jnp.float32),
    )(x, mi2, wh1a, wh1b0, wh1b1, bh1, wh2, bh2)


def kernel(X, E, emb_nodes, emb_edges, edge_index, We1, be1, We2, be2,
           Wh1, bh1, Wh2, bh2):
    f32 = jnp.float32
    X = X.astype(f32)
    src = edge_index[0].astype(jnp.int32)
    dst = edge_index[1].astype(jnp.int32)

    we1a = We1[:D]
    we1b = We1[D:2 * D]
    we1c = jnp.pad(We1[2 * D:2 * D + 3], ((0, 5), (0, 0)))
    we1d = We1[2 * D + 3:]
    em8 = jnp.pad(emb_nodes.astype(f32), ((0, 0), (0, 5)))

    p, q = _node_pq(X, em8, we1a, -we1c, we1b, we1c)
    ps, qd = _gather_pq(p, q, src, dst)
    mij2 = _edge_mlp(ps, qd, emb_edges.astype(f32), we1d, be1.reshape(1, D),
                     We2.astype(jnp.bfloat16), be2.reshape(1, D))
    zero = jnp.zeros((N_NODES, 128), f32)
    mi2 = _scatter_add(mij2, dst, zero)
    x_out = _node_mlp(X, mi2, Wh1[:D], Wh1[D:D + 128], Wh1[D + 128:],
                      bh1.reshape(1, D), Wh2, bh2.reshape(1, D))
    return (x_out, E, emb_nodes, emb_edges)


# speedup vs baseline: 4.5521x; 1.0089x over previous
import jax
import jax.numpy as jnp
from jax import lax
from jax.experimental import pallas as pl
from jax.experimental.pallas import tpu as pltpu
from jax.experimental.pallas import tpu_sc as plsc

N_NODES = 10000
N_EDGES = 160000
D = 256
DI = 128
NC, NS = 2, 16
NW = NC * NS
EPW = N_EDGES // NW
EPT = N_EDGES // NS
KCH = 5
ECH = N_EDGES // KCH
EPWC = ECH // NW
GCH = 128
SCH = 80
NSLOT = 3
SSLOT = 4


def _silu(x):
    return x * jax.nn.sigmoid(x)


def _pack_bf16(x):
    b = lax.bitcast_convert_type(x, jnp.uint32)
    r = b + jnp.uint32(0x7FFF) + ((b >> 16) & jnp.uint32(1))
    hi = r[:, :DI] & jnp.uint32(0xFFFF0000)
    lo = r[:, DI:] >> 16
    return lax.bitcast_convert_type(hi | lo, jnp.int32)


def _unpack_bf16(w_i32):
    w = lax.bitcast_convert_type(w_i32, jnp.uint32)
    hi = lax.bitcast_convert_type(w & jnp.uint32(0xFFFF0000), jnp.float32)
    lo = lax.bitcast_convert_type(w << 16, jnp.float32)
    return jnp.concatenate([hi, lo], axis=1)


def _pq_body(x_ref, em_ref, wa_ref, wcp_ref, wb_ref, wcq_ref, p_ref, q_ref):
    x = x_ref[...]
    em = em_ref[...]
    p = (jnp.dot(x, wa_ref[...], preferred_element_type=jnp.float32)
         + jnp.dot(em, wcp_ref[...], preferred_element_type=jnp.float32))
    q = (jnp.dot(x, wb_ref[...], preferred_element_type=jnp.float32)
         + jnp.dot(em, wcq_ref[...], preferred_element_type=jnp.float32))
    p_ref[...] = _pack_bf16(p)
    q_ref[...] = _pack_bf16(q)


def _node_pq(x, em8, wa, wcp, wb, wcq):
    blk = 2000
    return pl.pallas_call(
        _pq_body,
        grid=(N_NODES // blk,),
        in_specs=[
            pl.BlockSpec((blk, D), lambda i: (i, 0)),
            pl.BlockSpec((blk, 8), lambda i: (i, 0)),
            pl.BlockSpec((D, D), lambda i: (0, 0)),
            pl.BlockSpec((8, D), lambda i: (0, 0)),
            pl.BlockSpec((D, D), lambda i: (0, 0)),
            pl.BlockSpec((8, D), lambda i: (0, 0)),
        ],
        out_specs=[
            pl.BlockSpec((blk, DI), lambda i: (i, 0)),
            pl.BlockSpec((blk, DI), lambda i: (i, 0)),
        ],
        out_shape=[jax.ShapeDtypeStruct((N_NODES, DI), jnp.int32)] * 2,
    )(x, em8, wa, wcp, wb, wcq)


def _gather_body(p_hbm, q_hbm, src_hbm, dst_hbm, ps_hbm, qd_hbm,
                 idxs_v, idxd_v, pbuf, qbuf, semp, semq, semwp, semwq):
    c = lax.axis_index("c")
    s = lax.axis_index("s")
    w = s * NC + c
    base = pl.multiple_of(w * EPWC, 8)
    pltpu.sync_copy(src_hbm.at[pl.ds(base, EPWC)], idxs_v)
    pltpu.sync_copy(dst_hbm.at[pl.ds(base, EPWC)], idxd_v)

    nfull = EPWC // GCH
    tail = EPWC - nfull * GCH

    def issue(i, b):
        off = pl.multiple_of(i * GCH, 8)
        pltpu.async_copy(p_hbm.at[idxs_v.at[pl.ds(off, GCH)]],
                         pbuf.at[b], semp[b])
        pltpu.async_copy(q_hbm.at[idxd_v.at[pl.ds(off, GCH)]],
                         qbuf.at[b], semq[b])

    def wait_gather(b):
        pltpu.make_async_copy(p_hbm.at[idxs_v.at[pl.ds(0, GCH)]],
                              pbuf.at[b], semp[b]).wait()
        pltpu.make_async_copy(q_hbm.at[idxd_v.at[pl.ds(0, GCH)]],
                              qbuf.at[b], semq[b]).wait()

    def wb(i, b):
        off = pl.multiple_of(base + i * GCH, 8)
        pltpu.async_copy(pbuf.at[b], ps_hbm.at[pl.ds(off, GCH)], semwp[b])
        pltpu.async_copy(qbuf.at[b], qd_hbm.at[pl.ds(off, GCH)], semwq[b])

    def wait_wb(b):
        pltpu.make_async_copy(p_hbm.at[idxs_v.at[pl.ds(0, GCH)]],
                              pbuf.at[b], semwp[b]).wait()
        pltpu.make_async_copy(q_hbm.at[idxd_v.at[pl.ds(0, GCH)]],
                              qbuf.at[b], semwq[b]).wait()

    issue(0, 0)
    issue(1, 1)

    def step(i, b):
        wait_gather(b)
        wb(i, b)
        b2 = (b + 2) % NSLOT

        @pl.when(i >= 1)
        def _():
            wait_wb(b2)

        @pl.when(i + 2 < nfull)
        def _():
            issue(i + 2, b2)

    def outer(k, _):
        for b in range(NSLOT):
            step(NSLOT * k + b, b)
        return 0

    lax.fori_loop(0, nfull // NSLOT, outer, 0)
    for i in range(nfull - nfull % NSLOT, nfull):
        step(i, i % NSLOT)

    ts = (nfull - 2) % NSLOT
    toff = nfull * GCH
    cp1 = pltpu.async_copy(p_hbm.at[idxs_v.at[pl.ds(toff, tail)]],
                           pbuf.at[ts].at[pl.ds(0, tail)], semp[ts])
    cp2 = pltpu.async_copy(q_hbm.at[idxd_v.at[pl.ds(toff, tail)]],
                           qbuf.at[ts].at[pl.ds(0, tail)], semq[ts])
    cp1.wait()
    cp2.wait()
    pltpu.sync_copy(pbuf.at[ts].at[pl.ds(0, tail)],
                    ps_hbm.at[pl.ds(base + toff, tail)])
    pltpu.sync_copy(qbuf.at[ts].at[pl.ds(0, tail)],
                    qd_hbm.at[pl.ds(base + toff, tail)])
    wait_wb((nfull - 1) % NSLOT)


def _gather_pq(p, q, src, dst):
    mesh = plsc.VectorSubcoreMesh(core_axis_name="c", subcore_axis_name="s",
                                  num_cores=NC, num_subcores=NS)
    return pl.kernel(
        _gather_body,
        out_type=[jax.ShapeDtypeStruct((ECH, DI), jnp.int32)] * 2,
        mesh=mesh,
        scratch_types=[
            pltpu.VMEM((EPWC,), jnp.int32),
            pltpu.VMEM((EPWC,), jnp.int32),
            pltpu.VMEM((NSLOT, GCH, DI), jnp.int32),
            pltpu.VMEM((NSLOT, GCH, DI), jnp.int32),
            [pltpu.SemaphoreType.DMA] * NSLOT,
            [pltpu.SemaphoreType.DMA] * NSLOT,
            [pltpu.SemaphoreType.DMA] * NSLOT,
            [pltpu.SemaphoreType.DMA] * NSLOT,
        ],
    )(p, q, src, dst)


def _edge_body(ps_ref, qd_ref, e5_ref, w1d_ref, be1_ref, we2_ref, be2_ref,
               out_ref):
    pre = (_unpack_bf16(ps_ref[...]) + _unpack_bf16(qd_ref[...])
           + jnp.dot(e5_ref[...], w1d_ref[...],
                     preferred_element_type=jnp.float32)
           + be1_ref[...])
    h = _silu(pre)
    m = _silu(jnp.dot(h.astype(jnp.bfloat16), we2_ref[...],
                      preferred_element_type=jnp.float32)
              + be2_ref[...])
    out_ref[0] = m[:, :128]
    out_ref[1] = m[:, 128:]


def _edge_mlp(ps, qd, e5, w1d, be1, we2_bf, be2):
    blk = 1000
    n = ps.shape[0]
    return pl.pallas_call(
        _edge_body,
        grid=(n // blk,),
        in_specs=[
            pl.BlockSpec((blk, DI), lambda i: (i, 0)),
            pl.BlockSpec((blk, DI), lambda i: (i, 0)),
            pl.BlockSpec((blk, 5), lambda i: (i, 0)),
            pl.BlockSpec((5, D), lambda i: (0, 0)),
            pl.BlockSpec((1, D), lambda i: (0, 0)),
            pl.BlockSpec((D, D), lambda i: (0, 0)),
            pl.BlockSpec((1, D), lambda i: (0, 0)),
        ],
        out_specs=pl.BlockSpec((2, blk, 128), lambda i: (0, i, 0)),
        out_shape=jax.ShapeDtypeStruct((2, n, 128), jnp.float32),
    )(ps, qd, e5, w1d, be1, we2_bf, be2)


EPTC = ECH // NS


def _scatter_body(m0, m1, m2, m3, m4, dst_hbm, zero_hbm, out_hbm,
                  idx_v, buf, acc, semi, semd, sems):
    mijs = (m0, m1, m2, m3, m4)
    c = lax.axis_index("c")
    s = lax.axis_index("s")
    rpt = 624
    niter = EPTC // SCH

    roff = pl.multiple_of(s * rpt, 8)
    pltpu.sync_copy(zero_hbm.at[pl.ds(roff, rpt)], acc.at[pl.ds(roff, rpt)])

    @pl.when(s == NS - 1)
    def _():
        pltpu.sync_copy(zero_hbm.at[pl.ds(NS * rpt, N_NODES - NS * rpt)],
                        acc.at[pl.ds(NS * rpt, N_NODES - NS * rpt)])

    plsc.subcore_barrier()

    for a in range(KCH):
        mij = mijs[a]
        ebase = pl.multiple_of(a * ECH + s * EPTC, 8)

        def issue_load(j, b):
            off = pl.multiple_of(ebase + j * SCH, 8)
            moff = pl.multiple_of(s * EPTC + j * SCH, 8)
            pltpu.async_copy(dst_hbm.at[pl.ds(off, SCH)], idx_v.at[b],
                             semi[b])
            pltpu.async_copy(mij.at[c, pl.ds(moff, SCH)], buf.at[b], semd[b])

        def wait_load(b):
            pltpu.make_async_copy(dst_hbm.at[pl.ds(0, SCH)],
                                  idx_v.at[b], semi[b]).wait()
            pltpu.make_async_copy(mij.at[0, pl.ds(0, SCH)],
                                  buf.at[b], semd[b]).wait()

        def wait_scatter(b):
            pltpu.make_async_copy(mij.at[0, pl.ds(0, SCH)],
                                  buf.at[b], sems[b]).wait()

        issue_load(0, 0)

        def step(j, b):
            wait_load(b)
            pltpu.async_copy(buf.at[b], acc.at[idx_v.at[b]], sems[b],
                             add=True)
            nb = (b + 1) % SSLOT

            @pl.when(j >= SSLOT - 1)
            def _():
                wait_scatter(nb)

            @pl.when(j + 1 < niter)
            def _():
                issue_load(j + 1, nb)

        def outer(k, _):
            for b in range(SSLOT):
                step(SSLOT * k + b, b)
            return 0

        lax.fori_loop(0, niter // SSLOT, outer, 0)
        for j in range(niter - niter % SSLOT, niter):
            step(j, j % SSLOT)
        skip = niter % SSLOT
        for b in range(SSLOT):
            if b != skip:
                wait_scatter(b)

    plsc.subcore_barrier()
    pltpu.sync_copy(acc.at[pl.ds(roff, rpt)], out_hbm.at[c, pl.ds(roff, rpt)])

    @pl.when(s == NS - 1)
    def _():
        pltpu.sync_copy(acc.at[pl.ds(NS * rpt, N_NODES - NS * rpt)],
                        out_hbm.at[c, pl.ds(NS * rpt, N_NODES - NS * rpt)])


def _scatter_add(mijs, dst, zero):
    mesh = plsc.VectorSubcoreMesh(core_axis_name="c", subcore_axis_name="s",
                                  num_cores=NC, num_subcores=NS)
    return pl.kernel(
        _scatter_body,
        out_type=jax.ShapeDtypeStruct((2, N_NODES, 128), jnp.float32),
        mesh=mesh,
        scratch_types=[
            pltpu.VMEM((SSLOT, SCH), jnp.int32),
            pltpu.VMEM((SSLOT, SCH, 128), jnp.float32),
            pltpu.VMEM_SHARED((N_NODES, 128), jnp.float32),
            [pltpu.SemaphoreType.DMA] * SSLOT,
            [pltpu.SemaphoreType.DMA] * SSLOT,
            [pltpu.SemaphoreType.DMA] * SSLOT,
        ],
    )(*mijs, dst, zero)


def _node_body(x_ref, mi_ref, wa_ref, w0_ref, w1_ref, bh1_ref, wh2_ref,
               bh2_ref, out_ref):
    h2 = _silu(jnp.dot(x_ref[...], wa_ref[...], preferred_element_type=jnp.float32)
               + jnp.dot(mi_ref[0], w0_ref[...], preferred_element_type=jnp.float32)
               + jnp.dot(mi_ref[1], w1_ref[...], preferred_element_type=jnp.float32)
               + bh1_ref[...])
    out_ref[...] = (jnp.dot(h2, wh2_ref[...], preferred_element_type=jnp.float32)
                    + bh2_ref[...])


def _node_mlp(x, mi2, wh1a, wh1b0, wh1b1, bh1, wh2, bh2):
    blk = 2000
    return pl.pallas_call(
        _node_body,
        grid=(N_NODES // blk,),
        in_specs=[
            pl.BlockSpec((blk, D), lambda i: (i, 0)),
            pl.BlockSpec((2, blk, 128), lambda i: (0, i, 0)),
            pl.BlockSpec((D, D), lambda i: (0, 0)),
            pl.BlockSpec((128, D), lambda i: (0, 0)),
            pl.BlockSpec((128, D), lambda i: (0, 0)),
            pl.BlockSpec((1, D), lambda i: (0, 0)),
            pl.BlockSpec((D, D), lambda i: (0, 0)),
            pl.BlockSpec((1, D), lambda i: (0, 0)),
        ],
        out_specs=pl.BlockSpec((blk, D), lambda i: (i, 0)),
        out_shape=jax.ShapeDtypeStruct((N_NODES, D), jnp.float32),
    )(x, mi2, wh1a, wh1b0, wh1b1, bh1, wh2, bh2)


def kernel(X, E, emb_nodes, emb_edges, edge_index, We1, be1, We2, be2,
           Wh1, bh1, Wh2, bh2):
    f32 = jnp.float32
    X = X.astype(f32)
    src = edge_index[0].astype(jnp.int32)
    dst = edge_index[1].astype(jnp.int32)

    we1a = We1[:D]
    we1b = We1[D:2 * D]
    we1c = jnp.pad(We1[2 * D:2 * D + 3], ((0, 5), (0, 0)))
    we1d = We1[2 * D + 3:]
    em8 = jnp.pad(emb_nodes.astype(f32), ((0, 0), (0, 5)))

    p, q = _node_pq(X, em8, we1a, -we1c, we1b, we1c)
    e5 = emb_edges.astype(f32)
    we2b = We2.astype(jnp.bfloat16)
    be1r = be1.reshape(1, D)
    be2r = be2.reshape(1, D)
    mijs = []
    for a in range(KCH):
        sl = slice(a * ECH, (a + 1) * ECH)
        ps, qd = _gather_pq(p, q, src[sl], dst[sl])
        mijs.append(_edge_mlp(ps, qd, e5[sl], we1d, be1r, we2b, be2r))
    zero = jnp.zeros((N_NODES, 128), f32)
    mi2 = _scatter_add(mijs, dst, zero)
    x_out = _node_mlp(X, mi2, Wh1[:D], Wh1[D:D + 128], Wh1[D + 128:],
                      bh1.reshape(1, D), Wh2, bh2.reshape(1, D))
    return (x_out, E, emb_nodes, emb_edges)


# speedup vs baseline: 5.0256x; 1.1040x over previous
import jax
import jax.numpy as jnp
from jax import lax
from jax.experimental import pallas as pl
from jax.experimental.pallas import tpu as pltpu
from jax.experimental.pallas import tpu_sc as plsc

N_NODES = 10000
N_EDGES = 160000
D = 256
DI = 128
NC, NS = 2, 16
NW = NC * NS
EPW = N_EDGES // NW
EPT = N_EDGES // NS
KCH = 5
ECH = N_EDGES // KCH
EPWC = ECH // NW
GCH = 128
SCH = 80
NSLOT = 3
SSLOT = 4


def _silu(x):
    return x * jax.nn.sigmoid(x)


def _pack_bf16(x):
    b = lax.bitcast_convert_type(x, jnp.uint32)
    r = b + jnp.uint32(0x7FFF) + ((b >> 16) & jnp.uint32(1))
    hi = r[:, :DI] & jnp.uint32(0xFFFF0000)
    lo = r[:, DI:] >> 16
    return lax.bitcast_convert_type(hi | lo, jnp.int32)


def _unpack_bf16(w_i32):
    w = lax.bitcast_convert_type(w_i32, jnp.uint32)
    hi = lax.bitcast_convert_type(w & jnp.uint32(0xFFFF0000), jnp.float32)
    lo = lax.bitcast_convert_type(w << 16, jnp.float32)
    return jnp.concatenate([hi, lo], axis=1)


def _pq_body(x_ref, em_ref, wa_ref, wcp_ref, wb_ref, wcq_ref, p_ref, q_ref):
    x = x_ref[...]
    em = em_ref[...]
    p = (jnp.dot(x, wa_ref[...], preferred_element_type=jnp.float32)
         + jnp.dot(em, wcp_ref[...], preferred_element_type=jnp.float32))
    q = (jnp.dot(x, wb_ref[...], preferred_element_type=jnp.float32)
         + jnp.dot(em, wcq_ref[...], preferred_element_type=jnp.float32))
    p_ref[...] = _pack_bf16(p)
    q_ref[...] = _pack_bf16(q)


def _node_pq(x, em8, wa, wcp, wb, wcq):
    blk = 2000
    return pl.pallas_call(
        _pq_body,
        grid=(N_NODES // blk,),
        in_specs=[
            pl.BlockSpec((blk, D), lambda i: (i, 0)),
            pl.BlockSpec((blk, 8), lambda i: (i, 0)),
            pl.BlockSpec((D, D), lambda i: (0, 0)),
            pl.BlockSpec((8, D), lambda i: (0, 0)),
            pl.BlockSpec((D, D), lambda i: (0, 0)),
            pl.BlockSpec((8, D), lambda i: (0, 0)),
        ],
        out_specs=[
            pl.BlockSpec((blk, DI), lambda i: (i, 0)),
            pl.BlockSpec((blk, DI), lambda i: (i, 0)),
        ],
        out_shape=[jax.ShapeDtypeStruct((N_NODES, DI), jnp.int32)] * 2,
    )(x, em8, wa, wcp, wb, wcq)


def _gather_body(p_hbm, q_hbm, src_hbm, dst_hbm, ps_hbm, qd_hbm,
                 idxs_v, idxd_v, pbuf, qbuf, semp, semq, semwp, semwq):
    c = lax.axis_index("c")
    s = lax.axis_index("s")
    w = s * NC + c
    base = pl.multiple_of(w * EPWC, 8)
    pltpu.sync_copy(src_hbm.at[pl.ds(base, EPWC)], idxs_v)
    pltpu.sync_copy(dst_hbm.at[pl.ds(base, EPWC)], idxd_v)

    nfull = EPWC // GCH
    tail = EPWC - nfull * GCH

    def issue(i, b):
        off = pl.multiple_of(i * GCH, 8)
        pltpu.async_copy(p_hbm.at[idxs_v.at[pl.ds(off, GCH)]],
                         pbuf.at[b], semp[b])
        pltpu.async_copy(q_hbm.at[idxd_v.at[pl.ds(off, GCH)]],
                         qbuf.at[b], semq[b])

    def wait_gather(b):
        pltpu.make_async_copy(p_hbm.at[idxs_v.at[pl.ds(0, GCH)]],
                              pbuf.at[b], semp[b]).wait()
        pltpu.make_async_copy(q_hbm.at[idxd_v.at[pl.ds(0, GCH)]],
                              qbuf.at[b], semq[b]).wait()

    def wb(i, b):
        off = pl.multiple_of(base + i * GCH, 8)
        pltpu.async_copy(pbuf.at[b], ps_hbm.at[pl.ds(off, GCH)], semwp[b])
        pltpu.async_copy(qbuf.at[b], qd_hbm.at[pl.ds(off, GCH)], semwq[b])

    def wait_wb(b):
        pltpu.make_async_copy(p_hbm.at[idxs_v.at[pl.ds(0, GCH)]],
                              pbuf.at[b], semwp[b]).wait()
        pltpu.make_async_copy(q_hbm.at[idxd_v.at[pl.ds(0, GCH)]],
                              qbuf.at[b], semwq[b]).wait()

    issue(0, 0)
    issue(1, 1)

    def step(i, b):
        wait_gather(b)
        wb(i, b)
        b2 = (b + 2) % NSLOT

        @pl.when(i >= 1)
        def _():
            wait_wb(b2)

        @pl.when(i + 2 < nfull)
        def _():
            issue(i + 2, b2)

    def outer(k, _):
        for b in range(NSLOT):
            step(NSLOT * k + b, b)
        return 0

    lax.fori_loop(0, nfull // NSLOT, outer, 0)
    for i in range(nfull - nfull % NSLOT, nfull):
        step(i, i % NSLOT)

    ts = (nfull - 2) % NSLOT
    toff = nfull * GCH
    cp1 = pltpu.async_copy(p_hbm.at[idxs_v.at[pl.ds(toff, tail)]],
                           pbuf.at[ts].at[pl.ds(0, tail)], semp[ts])
    cp2 = pltpu.async_copy(q_hbm.at[idxd_v.at[pl.ds(toff, tail)]],
                           qbuf.at[ts].at[pl.ds(0, tail)], semq[ts])
    cp1.wait()
    cp2.wait()
    pltpu.sync_copy(pbuf.at[ts].at[pl.ds(0, tail)],
                    ps_hbm.at[pl.ds(base + toff, tail)])
    pltpu.sync_copy(qbuf.at[ts].at[pl.ds(0, tail)],
                    qd_hbm.at[pl.ds(base + toff, tail)])
    wait_wb((nfull - 1) % NSLOT)


def _gather_pq(p, q, src, dst):
    mesh = plsc.VectorSubcoreMesh(core_axis_name="c", subcore_axis_name="s",
                                  num_cores=NC, num_subcores=NS)
    return pl.kernel(
        _gather_body,
        out_type=[jax.ShapeDtypeStruct((ECH, DI), jnp.int32)] * 2,
        mesh=mesh,
        scratch_types=[
            pltpu.VMEM((EPWC,), jnp.int32),
            pltpu.VMEM((EPWC,), jnp.int32),
            pltpu.VMEM((NSLOT, GCH, DI), jnp.int32),
            pltpu.VMEM((NSLOT, GCH, DI), jnp.int32),
            [pltpu.SemaphoreType.DMA] * NSLOT,
            [pltpu.SemaphoreType.DMA] * NSLOT,
            [pltpu.SemaphoreType.DMA] * NSLOT,
            [pltpu.SemaphoreType.DMA] * NSLOT,
        ],
    )(p, q, src, dst)


def _edge_body(ps_ref, qd_ref, e5_ref, w1d_ref, be1_ref, we2_ref, be2_ref,
               out_ref):
    pre = (_unpack_bf16(ps_ref[...]) + _unpack_bf16(qd_ref[...])
           + jnp.dot(e5_ref[...], w1d_ref[...],
                     preferred_element_type=jnp.float32)
           + be1_ref[...])
    h = _silu(pre)
    m = _silu(jnp.dot(h.astype(jnp.bfloat16), we2_ref[...],
                      preferred_element_type=jnp.float32)
              + be2_ref[...])
    out_ref[0] = m[:, :128]
    out_ref[1] = m[:, 128:]


def _edge_mlp(ps, qd, e5, a, w1d, be1, we2_bf, be2):
    blk = 1000
    n = ps.shape[0]
    off = a * (n // blk)
    return pl.pallas_call(
        _edge_body,
        grid=(n // blk,),
        in_specs=[
            pl.BlockSpec((blk, DI), lambda i: (i, 0)),
            pl.BlockSpec((blk, DI), lambda i: (i, 0)),
            pl.BlockSpec((blk, 5), lambda i: (off + i, 0)),
            pl.BlockSpec((5, D), lambda i: (0, 0)),
            pl.BlockSpec((1, D), lambda i: (0, 0)),
            pl.BlockSpec((D, D), lambda i: (0, 0)),
            pl.BlockSpec((1, D), lambda i: (0, 0)),
        ],
        out_specs=pl.BlockSpec((2, blk, 128), lambda i: (0, i, 0)),
        out_shape=jax.ShapeDtypeStruct((2, n, 128), jnp.float32),
    )(ps, qd, e5, w1d, be1, we2_bf, be2)


EPTC = ECH // NS


def _make_scatter_body(nchunks, a0):
    def body(*refs):
        mijs = refs[:nchunks]
        dst_hbm, zero_hbm, out_hbm = refs[nchunks:nchunks + 3]
        idx_v, buf, acc, semi, semd, sems = refs[nchunks + 3:]
        c = lax.axis_index("c")
        s = lax.axis_index("s")
        rpt = 624
        niter = EPTC // SCH

        roff = pl.multiple_of(s * rpt, 8)
        pltpu.sync_copy(zero_hbm.at[pl.ds(roff, rpt)],
                        acc.at[pl.ds(roff, rpt)])

        @pl.when(s == NS - 1)
        def _():
            pltpu.sync_copy(zero_hbm.at[pl.ds(NS * rpt, N_NODES - NS * rpt)],
                            acc.at[pl.ds(NS * rpt, N_NODES - NS * rpt)])

        plsc.subcore_barrier()

        for a in range(nchunks):
            mij = mijs[a]
            ebase = pl.multiple_of((a0 + a) * ECH + s * EPTC, 8)

            def issue_load(j, b):
                off = pl.multiple_of(ebase + j * SCH, 8)
                moff = pl.multiple_of(s * EPTC + j * SCH, 8)
                pltpu.async_copy(dst_hbm.at[pl.ds(off, SCH)], idx_v.at[b],
                                 semi[b])
                pltpu.async_copy(mij.at[c, pl.ds(moff, SCH)], buf.at[b],
                                 semd[b])

            def wait_load(b):
                pltpu.make_async_copy(dst_hbm.at[pl.ds(0, SCH)],
                                      idx_v.at[b], semi[b]).wait()
                pltpu.make_async_copy(mij.at[0, pl.ds(0, SCH)],
                                      buf.at[b], semd[b]).wait()

            def wait_scatter(b):
                pltpu.make_async_copy(mij.at[0, pl.ds(0, SCH)],
                                      buf.at[b], sems[b]).wait()

            issue_load(0, 0)

            def step(j, b):
                wait_load(b)
                pltpu.async_copy(buf.at[b], acc.at[idx_v.at[b]], sems[b],
                                 add=True)
                nb = (b + 1) % SSLOT

                @pl.when(j >= SSLOT - 1)
                def _():
                    wait_scatter(nb)

                @pl.when(j + 1 < niter)
                def _():
                    issue_load(j + 1, nb)

            def outer(k, _):
                for b in range(SSLOT):
                    step(SSLOT * k + b, b)
                return 0

            lax.fori_loop(0, niter // SSLOT, outer, 0)
            for j in range(niter - niter % SSLOT, niter):
                step(j, j % SSLOT)
            skip = niter % SSLOT
            for b in range(SSLOT):
                if b != skip:
                    wait_scatter(b)

        plsc.subcore_barrier()
        pltpu.sync_copy(acc.at[pl.ds(roff, rpt)],
                        out_hbm.at[c, pl.ds(roff, rpt)])

        @pl.when(s == NS - 1)
        def _():
            pltpu.sync_copy(acc.at[pl.ds(NS * rpt, N_NODES - NS * rpt)],
                            out_hbm.at[c, pl.ds(NS * rpt, N_NODES - NS * rpt)])

    return body


def _scatter_add(mijs, a0, dst, zero):
    mesh = plsc.VectorSubcoreMesh(core_axis_name="c", subcore_axis_name="s",
                                  num_cores=NC, num_subcores=NS)
    return pl.kernel(
        _make_scatter_body(len(mijs), a0),
        out_type=jax.ShapeDtypeStruct((2, N_NODES, 128), jnp.float32),
        mesh=mesh,
        scratch_types=[
            pltpu.VMEM((SSLOT, SCH), jnp.int32),
            pltpu.VMEM((SSLOT, SCH, 128), jnp.float32),
            pltpu.VMEM_SHARED((N_NODES, 128), jnp.float32),
            [pltpu.SemaphoreType.DMA] * SSLOT,
            [pltpu.SemaphoreType.DMA] * SSLOT,
            [pltpu.SemaphoreType.DMA] * SSLOT,
        ],
    )(*mijs, dst, zero)


def _node_body(x_ref, mi_ref, wa_ref, w0_ref, w1_ref, bh1_ref, wh2_ref,
               bh2_ref, out_ref):
    h2 = _silu(jnp.dot(x_ref[...], wa_ref[...], preferred_element_type=jnp.float32)
               + jnp.dot(mi_ref[0], w0_ref[...], preferred_element_type=jnp.float32)
               + jnp.dot(mi_ref[1], w1_ref[...], preferred_element_type=jnp.float32)
               + bh1_ref[...])
    out_ref[...] = (jnp.dot(h2, wh2_ref[...], preferred_element_type=jnp.float32)
                    + bh2_ref[...])


def _node_mlp(x, mi2, wh1a, wh1b0, wh1b1, bh1, wh2, bh2):
    blk = 2000
    return pl.pallas_call(
        _node_body,
        grid=(N_NODES // blk,),
        in_specs=[
            pl.BlockSpec((blk, D), lambda i: (i, 0)),
            pl.BlockSpec((2, blk, 128), lambda i: (0, i, 0)),
            pl.BlockSpec((D, D), lambda i: (0, 0)),
            pl.BlockSpec((128, D), lambda i: (0, 0)),
            pl.BlockSpec((128, D), lambda i: (0, 0)),
            pl.BlockSpec((1, D), lambda i: (0, 0)),
            pl.BlockSpec((D, D), lambda i: (0, 0)),
            pl.BlockSpec((1, D), lambda i: (0, 0)),
        ],
        out_specs=pl.BlockSpec((blk, D), lambda i: (i, 0)),
        out_shape=jax.ShapeDtypeStruct((N_NODES, D), jnp.float32),
    )(x, mi2, wh1a, wh1b0, wh1b1, bh1, wh2, bh2)


def kernel(X, E, emb_nodes, emb_edges, edge_index, We1, be1, We2, be2,
           Wh1, bh1, Wh2, bh2):
    f32 = jnp.float32
    X = X.astype(f32)
    src = edge_index[0].astype(jnp.int32)
    dst = edge_index[1].astype(jnp.int32)

    we1a = We1[:D]
    we1b = We1[D:2 * D]
    we1c = jnp.pad(We1[2 * D:2 * D + 3], ((0, 5), (0, 0)))
    we1d = We1[2 * D + 3:]
    em8 = jnp.pad(emb_nodes.astype(f32), ((0, 0), (0, 5)))

    p, q = _node_pq(X, em8, we1a, -we1c, we1b, we1c)
    e5 = emb_edges.astype(f32)
    we2b = We2.astype(jnp.bfloat16)
    be1r = be1.reshape(1, D)
    be2r = be2.reshape(1, D)
    mijs = []
    for a in range(KCH):
        sl = slice(a * ECH, (a + 1) * ECH)
        ps, qd = _gather_pq(p, q, src[sl], dst[sl])
        mijs.append(_edge_mlp(ps, qd, e5, a, we1d, be1r, we2b, be2r))
    zero = jnp.zeros((N_NODES, 128), f32)
    mi2 = (_scatter_add(mijs[:3], 0, dst, zero)
           + _scatter_add(mijs[3:], 3, dst, zero))
    x_out = _node_mlp(X, mi2, Wh1[:D], Wh1[D:D + 128], Wh1[D + 128:],
                      bh1.reshape(1, D), Wh2, bh2.reshape(1, D))
    return (x_out, E, emb_nodes, emb_edges)
